# baseline (speedup 1.0000x reference)
import functools

import jax
import jax.numpy as jnp
from jax import lax
from jax.experimental import pallas as pl
from jax.experimental.pallas import tpu as pltpu
from jax.experimental.pallas import tpu_sc as plsc

_NC = 2
_NS = 16
_NW = _NC * _NS


def _make_sc_broadcast(B, S, H, chunk):
    rows_per_w = S // _NW
    n_chunks = rows_per_w // chunk
    mesh = plsc.VectorSubcoreMesh(core_axis_name="c", subcore_axis_name="s")

    @functools.partial(
        pl.kernel,
        mesh=mesh,
        out_type=jax.ShapeDtypeStruct((B, S, H), jnp.float32),
        scratch_types=[
            pltpu.VMEM((chunk, H), jnp.float32),
            pltpu.SemaphoreType.DMA,
        ],
    )
    def sc_broadcast(tab_hbm, out_hbm, buf, sem):
        wid = lax.axis_index("s") * _NC + lax.axis_index("c")
        base = wid * rows_per_w

        def step(j, carry):
            r0 = base + j * chunk
            pltpu.sync_copy(tab_hbm.at[pl.ds(r0, chunk)], buf)
            copies = [
                pltpu.async_copy(buf, out_hbm.at[b, pl.ds(r0, chunk)], sem)
                for b in range(B)
            ]
            for c in copies:
                c.wait()
            return carry

        lax.fori_loop(0, n_chunks, step, 0)

    return sc_broadcast


def kernel(inputs, position_embeddings):
    B, S, H = inputs.shape
    table = position_embeddings[:S]
    return _make_sc_broadcast(B, S, H, chunk=64)(table)
